# baseline (speedup 1.0000x reference)
import functools

import jax
import jax.numpy as jnp
import numpy as np
from jax import lax
from jax.experimental import pallas as pl
from jax.experimental.pallas import tpu as pltpu
from jax.experimental.pallas import tpu_sc as plsc

B = 4096
EMB = 128
VOCAB = 22
PAD_IDX = 0
PEP_PAD = 3
PEP_LEN = 27
MHC_LEN = 34

NC = 2
NS = 16
NW = NC * NS

PEP_TOK = B * PEP_LEN
MHC_TOK = B * MHC_LEN
PEP_PER_TILE = PEP_TOK // NW
MHC_PER_TILE = MHC_TOK // NW
CHUNK = 128
PEP_PAD_LEN = 32
MHC_PAD_LEN = 40
BPT = B // NW
PEP_PAD_PER_TILE = BPT * PEP_PAD_LEN
MHC_PAD_PER_TILE = BPT * MHC_PAD_LEN


def _pos_enc(length, emb):
    position = np.arange(length).reshape(-1, 1).astype(np.float32)
    div_term = np.exp(
        np.arange(0, emb, 2).astype(np.float32) * -(np.log(10000.0) / emb))
    pe = np.zeros((length, emb), dtype=np.float32)
    pe[:, 0::2] = np.sin(position * div_term)
    pe[:, 1::2] = np.cos(position * div_term)
    return pe


def _pep_posext():
    pe = np.zeros((PEP_LEN, EMB), dtype=np.float32)
    pe[PEP_PAD:PEP_PAD + 21] = _pos_enc(21, EMB)
    return pe


_PEP_POSEXT = _pep_posext()
_MHC_POS = _pos_enc(MHC_LEN, EMB)


def _prep_body(pw_ref, mw_ref, px_ref, pe_ref, me_ref,
               tpep_ref, tmhc_ref, mask_ref):
    vnz = (lax.broadcasted_iota(jnp.int32, (VOCAB, EMB), 0) != PAD_IDX)
    vnz = vnz.astype(jnp.float32)
    tpep_ref[...] = (pw_ref[...][None, :, :]
                     + pe_ref[...][:, None, :] * vnz[None, :, :])
    tmhc_ref[...] = mw_ref[...][None, :, :] + me_ref[...][:, None, :]
    mask_ref[...] = px_ref[:, PEP_PAD:PEP_PAD + 21] != PAD_IDX


_prep = pl.pallas_call(
    _prep_body,
    out_shape=(
        jax.ShapeDtypeStruct((PEP_LEN, VOCAB, EMB), jnp.float32),
        jax.ShapeDtypeStruct((MHC_LEN, VOCAB, EMB), jnp.float32),
        jax.ShapeDtypeStruct((B, 21), jnp.bool_),
    ),
)


def _sc_body(tpep_hbm, tmhc_hbm, px_hbm, mx_hbm,
             pout_hbm, mout_hbm, x_v, idx_v, rows_v,
             sga, sgb, soa, sob):
    wid = lax.axis_index("s") * NC + lax.axis_index("c")
    lane = lax.iota(jnp.int32, 16)

    def build_idx_pep():
        off_e = lane * VOCAB
        off_o = jnp.where(lane < 11, (16 + lane) * VOCAB, 26 * VOCAB)

        def row(m, carry):
            xe = x_v[pl.ds(m * PEP_LEN, 16)]
            idx_v[pl.ds(m * PEP_PAD_LEN, 16)] = xe + off_e
            xo = x_v[pl.ds(m * PEP_LEN + 16, 16)]
            xo = jnp.where(lane < 11, xo, 0)
            idx_v[pl.ds(m * PEP_PAD_LEN + 16, 16)] = xo + off_o
            return carry

        lax.fori_loop(0, BPT, row, 0)

    def build_idx_mhc():
        off_g = [
            lane * VOCAB,
            (16 + lane) * VOCAB,
            jnp.where(lane < 2, (32 + lane) * VOCAB,
                      jnp.where(lane >= 8, (lane - 8) * VOCAB, 33 * VOCAB)),
            (8 + lane) * VOCAB,
            jnp.where(lane < 10, (24 + lane) * VOCAB, 33 * VOCAB),
        ]

        def pair(q, carry):
            s = 2 * q * MHC_LEN
            d = 2 * q * MHC_PAD_LEN
            x0 = x_v[pl.ds(s, 16)]
            idx_v[pl.ds(d, 16)] = x0 + off_g[0]
            x1 = x_v[pl.ds(s + 16, 16)]
            idx_v[pl.ds(d + 16, 16)] = x1 + off_g[1]
            xa = x_v[pl.ds(s + 32, 16)]
            xb = x_v[pl.ds(s + 26, 16)]
            x2 = jnp.where(lane < 2, xa, jnp.where(lane >= 8, xb, 0))
            idx_v[pl.ds(d + 32, 16)] = x2 + off_g[2]
            x3 = x_v[pl.ds(s + 42, 16)]
            idx_v[pl.ds(d + 48, 16)] = x3 + off_g[3]
            x4 = x_v[pl.ds(s + 58, 16)]
            x4 = jnp.where(lane < 10, x4, 0)
            idx_v[pl.ds(d + 64, 16)] = x4 + off_g[4]
            return carry

        lax.fori_loop(0, BPT // 2, pair, 0)

    def run_table(x_hbm, tbl_hbm, out_hbm, ntok, npad, chunk, build_idx):
        n = npad // chunk
        nit = n // 2
        base = wid * npad
        pltpu.sync_copy(x_hbm.at[pl.ds(wid * ntok, ntok)],
                        x_v.at[pl.ds(0, ntok)])
        build_idx()

        buf_a = rows_v.at[0, pl.ds(0, chunk)]
        buf_b = rows_v.at[1, pl.ds(0, chunk)]

        def gstart(c, buf, sem):
            pltpu.async_copy(
                tbl_hbm.at[idx_v.at[pl.ds(c * chunk, chunk)]], buf, sem)

        def gwait(buf, sem):
            pltpu.make_async_copy(
                tbl_hbm.at[idx_v.at[pl.ds(0, chunk)]], buf, sem).wait()

        def ostart(c, buf, sem):
            pltpu.async_copy(buf, out_hbm.at[pl.ds(base + c * chunk, chunk)],
                             sem)

        def owait(buf, sem):
            pltpu.make_async_copy(buf, out_hbm.at[pl.ds(base, chunk)],
                                  sem).wait()

        gstart(0, buf_a, sga)
        gstart(1, buf_b, sgb)

        def body(i, carry):
            c = 2 * i
            gwait(buf_a, sga)
            ostart(c, buf_a, soa)
            gwait(buf_b, sgb)
            ostart(c + 1, buf_b, sob)

            @pl.when(i < nit - 1)
            def _():
                owait(buf_a, soa)
                gstart(c + 2, buf_a, sga)
                owait(buf_b, sob)
                gstart(c + 3, buf_b, sgb)

            return carry

        lax.fori_loop(0, nit, body, 0)
        owait(buf_a, soa)
        owait(buf_b, sob)

    run_table(px_hbm, tpep_hbm, pout_hbm,
              PEP_PER_TILE, PEP_PAD_PER_TILE, CHUNK, build_idx_pep)
    run_table(mx_hbm, tmhc_hbm, mout_hbm,
              MHC_PER_TILE, MHC_PAD_PER_TILE, CHUNK, build_idx_mhc)


@functools.cache
def _make_gather():
    return pl.kernel(
        _sc_body,
        out_type=(
            jax.ShapeDtypeStruct((B * PEP_PAD_LEN, EMB), jnp.float32),
            jax.ShapeDtypeStruct((B * MHC_PAD_LEN, EMB), jnp.float32),
        ),
        mesh=plsc.VectorSubcoreMesh(core_axis_name="c", subcore_axis_name="s"),
        scratch_types=[
            pltpu.VMEM((MHC_PER_TILE + 32,), jnp.int32),
            pltpu.VMEM((MHC_PAD_PER_TILE,), jnp.int32),
            pltpu.VMEM((2, CHUNK, EMB), jnp.float32),
            pltpu.SemaphoreType.DMA,
            pltpu.SemaphoreType.DMA,
            pltpu.SemaphoreType.DMA,
            pltpu.SemaphoreType.DMA,
        ],
    )


@jax.jit
def kernel(peptide_x, mhc_x, peptide_W, mhc_W):
    px = peptide_x.astype(jnp.int32)
    mx = mhc_x.astype(jnp.int32)
    tpep3, tmhc3, masks = _prep(peptide_W, mhc_W, px,
                                jnp.asarray(_PEP_POSEXT), jnp.asarray(_MHC_POS))
    tpep = tpep3.reshape(PEP_LEN * VOCAB, EMB)
    tmhc = tmhc3.reshape(MHC_LEN * VOCAB, EMB)
    pout, mout = _make_gather()(tpep, tmhc, px.reshape(-1), mx.reshape(-1))
    return (pout.reshape(B, PEP_PAD_LEN, EMB)[:, :PEP_LEN, :],
            mout.reshape(B, MHC_PAD_LEN, EMB)[:, :MHC_LEN, :],
            masks)


# speedup vs baseline: 5.2125x; 5.2125x over previous
import functools

import jax
import jax.numpy as jnp
import numpy as np
from jax import lax
from jax.experimental import pallas as pl
from jax.experimental.pallas import tpu as pltpu
from jax.experimental.pallas import tpu_sc as plsc
from jax.experimental import layout as jex_layout

B = 4096
EMB = 128
VOCAB = 22
PAD_IDX = 0
PEP_PAD = 3
PEP_LEN = 27
MHC_LEN = 34

NC = 2
NS = 16
NW = NC * NS

PEP_TOK = B * PEP_LEN
MHC_TOK = B * MHC_LEN
PEP_PER_TILE = PEP_TOK // NW
MHC_PER_TILE = MHC_TOK // NW
CHUNK = 128


def _pos_enc(length, emb):
    position = np.arange(length).reshape(-1, 1).astype(np.float32)
    div_term = np.exp(
        np.arange(0, emb, 2).astype(np.float32) * -(np.log(10000.0) / emb))
    pe = np.zeros((length, emb), dtype=np.float32)
    pe[:, 0::2] = np.sin(position * div_term)
    pe[:, 1::2] = np.cos(position * div_term)
    return pe


def _pep_posext():
    pe = np.zeros((PEP_LEN, EMB), dtype=np.float32)
    pe[PEP_PAD:PEP_PAD + 21] = _pos_enc(21, EMB)
    return pe


_PEP_POSEXT = _pep_posext()
_MHC_POS = _pos_enc(MHC_LEN, EMB)
_OFFS_PEP = np.tile(
    np.arange(PEP_LEN, dtype=np.int32) * VOCAB, PEP_PER_TILE // PEP_LEN)
_OFFS_MHC = np.tile(
    np.arange(MHC_LEN, dtype=np.int32) * VOCAB, MHC_PER_TILE // MHC_LEN)


def _prep_body(pw_ref, mw_ref, px_ref, pe_ref, me_ref,
               tpep_ref, tmhc_ref, mask_ref):
    vnz = (lax.broadcasted_iota(jnp.int32, (VOCAB, EMB), 0) != PAD_IDX)
    vnz = vnz.astype(jnp.float32)
    tpep_ref[...] = (pw_ref[...][None, :, :]
                     + pe_ref[...][:, None, :] * vnz[None, :, :])
    tmhc_ref[...] = mw_ref[...][None, :, :] + me_ref[...][:, None, :]
    mask_ref[...] = px_ref[:, PEP_PAD:PEP_PAD + 21] != PAD_IDX


_prep = pl.pallas_call(
    _prep_body,
    out_shape=(
        jax.ShapeDtypeStruct((PEP_LEN, VOCAB, EMB), jnp.float32),
        jax.ShapeDtypeStruct((MHC_LEN, VOCAB, EMB), jnp.float32),
        jax.ShapeDtypeStruct((B, 21), jnp.bool_),
    ),
)


def _sc_body(tpep_hbm, tmhc_hbm, px_hbm, mx_hbm, offp_hbm, offm_hbm,
             pout_hbm, mout_hbm, x_v, idx_v, off_v, rows_v,
             sga, sgb, soa, sob):
    wid = lax.axis_index("s") * NC + lax.axis_index("c")

    def run_table(x_hbm, off_hbm, tbl_hbm, out_hbm, ntok, chunk):
        n = ntok // chunk
        nit = n // 2
        base = wid * ntok
        pltpu.sync_copy(x_hbm.at[pl.ds(base, ntok)], x_v.at[pl.ds(0, ntok)])
        pltpu.sync_copy(off_hbm, off_v.at[pl.ds(0, ntok)])

        def add_body(i, carry):
            s = pl.ds(i * 16, 16)
            idx_v[s] = x_v[s] + off_v[s]
            return carry

        lax.fori_loop(0, ntok // 16, add_body, 0)

        buf_a = rows_v.at[0, pl.ds(0, chunk)]
        buf_b = rows_v.at[1, pl.ds(0, chunk)]

        def gstart(c, buf, sem):
            pltpu.async_copy(
                tbl_hbm.at[idx_v.at[pl.ds(c * chunk, chunk)]], buf, sem)

        def gwait(buf, sem):
            pltpu.make_async_copy(
                tbl_hbm.at[idx_v.at[pl.ds(0, chunk)]], buf, sem).wait()

        def ostart(c, buf, sem):
            pltpu.async_copy(buf, out_hbm.at[pl.ds(base + c * chunk, chunk)],
                             sem)

        def owait(buf, sem):
            pltpu.make_async_copy(buf, out_hbm.at[pl.ds(base, chunk)],
                                  sem).wait()

        gstart(0, buf_a, sga)
        gstart(1, buf_b, sgb)

        def body(i, carry):
            c = 2 * i
            gwait(buf_a, sga)
            ostart(c, buf_a, soa)
            gwait(buf_b, sgb)
            ostart(c + 1, buf_b, sob)

            @pl.when(i < nit - 1)
            def _():
                owait(buf_a, soa)
                gstart(c + 2, buf_a, sga)
                owait(buf_b, sob)
                gstart(c + 3, buf_b, sgb)

            return carry

        lax.fori_loop(0, nit, body, 0)
        owait(buf_a, soa)
        owait(buf_b, sob)

    run_table(px_hbm, offp_hbm, tpep_hbm, pout_hbm, PEP_PER_TILE, 64)
    run_table(mx_hbm, offm_hbm, tmhc_hbm, mout_hbm, MHC_PER_TILE, CHUNK)


@functools.cache
def _make_gather():
    return pl.kernel(
        _sc_body,
        out_type=(
            jax.ShapeDtypeStruct((PEP_TOK, EMB), jnp.float32),
            jax.ShapeDtypeStruct((MHC_TOK, EMB), jnp.float32),
        ),
        mesh=plsc.VectorSubcoreMesh(core_axis_name="c", subcore_axis_name="s"),
        scratch_types=[
            pltpu.VMEM((MHC_PER_TILE,), jnp.int32),
            pltpu.VMEM((MHC_PER_TILE,), jnp.int32),
            pltpu.VMEM((MHC_PER_TILE,), jnp.int32),
            pltpu.VMEM((2, CHUNK, EMB), jnp.float32),
            pltpu.SemaphoreType.DMA,
            pltpu.SemaphoreType.DMA,
            pltpu.SemaphoreType.DMA,
            pltpu.SemaphoreType.DMA,
        ],
    )


def _kernel_impl(peptide_x, mhc_x, peptide_W, mhc_W):
    px = peptide_x.astype(jnp.int32)
    mx = mhc_x.astype(jnp.int32)
    tpep3, tmhc3, masks = _prep(peptide_W, mhc_W, px,
                                jnp.asarray(_PEP_POSEXT), jnp.asarray(_MHC_POS))
    tpep = tpep3.reshape(PEP_LEN * VOCAB, EMB)
    tmhc = tmhc3.reshape(MHC_LEN * VOCAB, EMB)
    pout, mout = _make_gather()(tpep, tmhc, px.reshape(-1), mx.reshape(-1),
                                jnp.asarray(_OFFS_PEP), jnp.asarray(_OFFS_MHC))
    return (pout.reshape(B, PEP_LEN, EMB),
            mout.reshape(B, MHC_LEN, EMB),
            masks)


@functools.cache
def _jitted_kernel():
    dev = jax.devices()[0]
    sharding = jax.sharding.SingleDeviceSharding(dev)
    dense3 = jex_layout.Format(
        jex_layout.Layout(major_to_minor=(0, 1, 2), tiling=((1, 128),)),
        sharding)
    return jax.jit(_kernel_impl, out_shardings=(dense3, dense3, None))


def kernel(peptide_x, mhc_x, peptide_W, mhc_W):
    return _jitted_kernel()(peptide_x, mhc_x, peptide_W, mhc_W)


# speedup vs baseline: 5.2190x; 1.0012x over previous
import functools

import jax
import jax.numpy as jnp
import numpy as np
from jax import lax
from jax.experimental import pallas as pl
from jax.experimental.pallas import tpu as pltpu
from jax.experimental.pallas import tpu_sc as plsc
from jax.experimental import layout as jex_layout

B = 4096
EMB = 128
VOCAB = 22
PAD_IDX = 0
PEP_PAD = 3
PEP_LEN = 27
MHC_LEN = 34

NC = 2
NS = 16
NW = NC * NS

PEP_TOK = B * PEP_LEN
MHC_TOK = B * MHC_LEN
PEP_PER_TILE = PEP_TOK // NW
MHC_PER_TILE = MHC_TOK // NW
CHUNK = 128


def _pos_enc(length, emb):
    position = np.arange(length).reshape(-1, 1).astype(np.float32)
    div_term = np.exp(
        np.arange(0, emb, 2).astype(np.float32) * -(np.log(10000.0) / emb))
    pe = np.zeros((length, emb), dtype=np.float32)
    pe[:, 0::2] = np.sin(position * div_term)
    pe[:, 1::2] = np.cos(position * div_term)
    return pe


def _pep_posext():
    pe = np.zeros((PEP_LEN, EMB), dtype=np.float32)
    pe[PEP_PAD:PEP_PAD + 21] = _pos_enc(21, EMB)
    return pe


_PEP_POSEXT = _pep_posext()
_MHC_POS = _pos_enc(MHC_LEN, EMB)
_OFFS_PEP = np.tile(
    np.arange(PEP_LEN, dtype=np.int32) * VOCAB, PEP_PER_TILE // PEP_LEN)
_OFFS_MHC = np.tile(
    np.arange(MHC_LEN, dtype=np.int32) * VOCAB, MHC_PER_TILE // MHC_LEN)


def _prep_body(pw_ref, mw_ref, px_ref, pe_ref, me_ref,
               tpep_ref, tmhc_ref, mask_ref):
    vnz = (lax.broadcasted_iota(jnp.int32, (VOCAB, EMB), 0) != PAD_IDX)
    vnz = vnz.astype(jnp.float32)
    tpep_ref[...] = (pw_ref[...][None, :, :]
                     + pe_ref[...][:, None, :] * vnz[None, :, :])
    tmhc_ref[...] = mw_ref[...][None, :, :] + me_ref[...][:, None, :]
    mask_ref[...] = px_ref[:, PEP_PAD:PEP_PAD + 21] != PAD_IDX


_prep = pl.pallas_call(
    _prep_body,
    out_shape=(
        jax.ShapeDtypeStruct((PEP_LEN, VOCAB, EMB), jnp.float32),
        jax.ShapeDtypeStruct((MHC_LEN, VOCAB, EMB), jnp.float32),
        jax.ShapeDtypeStruct((B, 21), jnp.bool_),
    ),
)


def _sc_body(tpep_hbm, tmhc_hbm, px_hbm, mx_hbm, offp_hbm, offm_hbm,
             pout_hbm, mout_hbm, x_v, idx_v, off_v, rows_v,
             sga, sgb, soa, sob):
    wid = lax.axis_index("s") * NC + lax.axis_index("c")

    def run_table(x_hbm, off_hbm, tbl_hbm, out_hbm, ntok, chunk):
        n = ntok // chunk
        nit = n // 2
        base = wid * ntok
        pltpu.sync_copy(x_hbm.at[pl.ds(base, ntok)], x_v.at[pl.ds(0, ntok)])
        pltpu.sync_copy(off_hbm, off_v.at[pl.ds(0, ntok)])

        def add_body(i, carry):
            s = pl.ds(i * 16, 16)
            idx_v[s] = x_v[s] + off_v[s]
            return carry

        lax.fori_loop(0, ntok // 16, add_body, 0)

        buf_a = rows_v.at[0, pl.ds(0, chunk)]
        buf_b = rows_v.at[1, pl.ds(0, chunk)]

        def gstart(c, buf, sem):
            pltpu.async_copy(
                tbl_hbm.at[idx_v.at[pl.ds(c * chunk, chunk)]], buf, sem)

        def gwait(buf, sem):
            pltpu.make_async_copy(
                tbl_hbm.at[idx_v.at[pl.ds(0, chunk)]], buf, sem).wait()

        def ostart(c, buf, sem):
            pltpu.async_copy(buf, out_hbm.at[pl.ds(base + c * chunk, chunk)],
                             sem)

        def owait(buf, sem):
            pltpu.make_async_copy(buf, out_hbm.at[pl.ds(base, chunk)],
                                  sem).wait()

        gstart(0, buf_a, sga)
        gstart(1, buf_b, sgb)

        def body(i, carry):
            c = 2 * i
            gwait(buf_a, sga)
            ostart(c, buf_a, soa)
            gwait(buf_b, sgb)
            ostart(c + 1, buf_b, sob)

            @pl.when(i < nit - 1)
            def _():
                owait(buf_a, soa)
                gstart(c + 2, buf_a, sga)
                owait(buf_b, sob)
                gstart(c + 3, buf_b, sgb)

            return carry

        lax.fori_loop(0, nit, body, 0)
        owait(buf_a, soa)
        owait(buf_b, sob)

    run_table(px_hbm, offp_hbm, tpep_hbm, pout_hbm, PEP_PER_TILE, 64)
    run_table(mx_hbm, offm_hbm, tmhc_hbm, mout_hbm, MHC_PER_TILE, CHUNK)


@functools.cache
def _make_gather():
    return pl.kernel(
        _sc_body,
        out_type=(
            jax.ShapeDtypeStruct((PEP_TOK, EMB), jnp.float32),
            jax.ShapeDtypeStruct((MHC_TOK, EMB), jnp.float32),
        ),
        mesh=plsc.VectorSubcoreMesh(core_axis_name="c", subcore_axis_name="s"),
        scratch_types=[
            pltpu.VMEM((MHC_PER_TILE,), jnp.int32),
            pltpu.VMEM((MHC_PER_TILE,), jnp.int32),
            pltpu.VMEM((MHC_PER_TILE,), jnp.int32),
            pltpu.VMEM((2, CHUNK, EMB), jnp.float32),
            pltpu.SemaphoreType.DMA,
            pltpu.SemaphoreType.DMA,
            pltpu.SemaphoreType.DMA,
            pltpu.SemaphoreType.DMA,
        ],
    )


def _kernel_impl(peptide_x, mhc_x, peptide_W, mhc_W):
    px = peptide_x.astype(jnp.int32)
    mx = mhc_x.astype(jnp.int32)
    tpep3, tmhc3, masks = _prep(peptide_W, mhc_W, px,
                                jnp.asarray(_PEP_POSEXT), jnp.asarray(_MHC_POS))
    tpep = tpep3.reshape(PEP_LEN * VOCAB, EMB)
    tmhc = tmhc3.reshape(MHC_LEN * VOCAB, EMB)
    pout, mout = _make_gather()(tpep, tmhc, px.reshape(-1), mx.reshape(-1),
                                jnp.asarray(_OFFS_PEP), jnp.asarray(_OFFS_MHC))
    pout3 = pout.reshape(B, PEP_LEN, EMB)
    mout3 = mout.reshape(B, MHC_LEN, EMB)
    dense3 = jex_layout.Layout(major_to_minor=(0, 1, 2), tiling=((1, 128),))
    pout3 = jex_layout.with_layout_constraint(pout3, dense3)
    mout3 = jex_layout.with_layout_constraint(mout3, dense3)
    return (pout3, mout3, masks)


@functools.cache
def _dense3_format():
    sharding = jax.sharding.SingleDeviceSharding(jax.devices()[0])
    return jex_layout.Format(
        jex_layout.Layout(major_to_minor=(0, 1, 2), tiling=((1, 128),)),
        sharding)


@functools.cache
def _jitted_kernel():
    dev = jax.devices()[0]
    sharding = jax.sharding.SingleDeviceSharding(dev)
    dense3 = jex_layout.Format(
        jex_layout.Layout(major_to_minor=(0, 1, 2), tiling=((1, 128),)),
        sharding)
    return jax.jit(_kernel_impl, out_shardings=(dense3, dense3, None))


def kernel(peptide_x, mhc_x, peptide_W, mhc_W):
    return _jitted_kernel()(peptide_x, mhc_x, peptide_W, mhc_W)
